# baseline (speedup 1.0000x reference)
import functools

import jax
import jax.numpy as jnp
from jax import lax
from jax.experimental import pallas as pl
from jax.experimental.pallas import tpu as pltpu
from jax.experimental.pallas import tpu_sc as plsc

NUM_EXPERTS = 8
TOP_K = 2
DIM = 1024
FFN = 2048
T = 2048
TT = TOP_K * T

BLK = 128
PADT = TT + NUM_EXPERTS * BLK
NB = PADT // BLK

NW = 32
CH_D = TT // NW
CH_C = T // NW


def _router_body(x_ref, gw_ref, sc_ref, ex_ref):
    x = x_ref[...]
    gw = gw_ref[...]
    l = lax.dot_general(x, gw, (((1,), (1,)), ((), ())),
                        preferred_element_type=jnp.float32)
    idx = lax.broadcasted_iota(jnp.int32, l.shape, 1)
    m1 = jnp.max(l, axis=1, keepdims=True)
    a1 = jnp.min(jnp.where(l >= m1, idx, NUM_EXPERTS), axis=1)
    neg = jnp.where(idx == a1[:, None], -jnp.inf, l)
    m2 = jnp.max(neg, axis=1, keepdims=True)
    a2 = jnp.min(jnp.where(neg >= m2, idx, NUM_EXPERTS), axis=1)
    den = jnp.sum(jnp.exp(l - m1), axis=1, keepdims=True)
    s1 = 1.0 / den
    s2 = jnp.exp(m2 - m1) / den
    sc_ref[...] = jnp.concatenate([s1, s2], axis=1)
    ex_ref[...] = jnp.concatenate([a1[:, None], a2[:, None]], axis=1)


def _router(x, gate_w):
    blk = 256
    return pl.pallas_call(
        _router_body,
        grid=(T // blk,),
        in_specs=[
            pl.BlockSpec((blk, DIM), lambda b: (b, 0)),
            pl.BlockSpec((NUM_EXPERTS, DIM), lambda b: (0, 0)),
        ],
        out_specs=[
            pl.BlockSpec((blk, TOP_K), lambda b: (b, 0)),
            pl.BlockSpec((blk, TOP_K), lambda b: (b, 0)),
        ],
        out_shape=[
            jax.ShapeDtypeStruct((T, TOP_K), jnp.float32),
            jax.ShapeDtypeStruct((T, TOP_K), jnp.int32),
        ],
    )(x, gate_w)


def _dispatch_body(x_hbm, tok_hbm, pos_hbm, disp_hbm, tokv, posv, rows, sem):
    w = lax.axis_index("s") * 2 + lax.axis_index("c")
    pltpu.sync_copy(tok_hbm.at[w], tokv)
    pltpu.sync_copy(pos_hbm.at[w], posv)
    for j in range(2):
        pltpu.async_copy(x_hbm.at[tokv.at[j]], rows, sem).wait()
        pltpu.async_copy(rows, disp_hbm.at[posv.at[j]], sem).wait()


def _dispatch_sc(x, tok, pos):
    h = CH_D // 2
    k = functools.partial(
        pl.kernel,
        mesh=plsc.VectorSubcoreMesh(core_axis_name="c", subcore_axis_name="s"),
        out_type=jax.ShapeDtypeStruct((PADT, DIM), jnp.float32),
        scratch_types=[
            pltpu.VMEM((2, h), jnp.int32),
            pltpu.VMEM((2, h), jnp.int32),
            pltpu.VMEM((h, DIM), jnp.float32),
            pltpu.SemaphoreType.DMA,
        ],
    )(_dispatch_body)
    return k(x, tok.reshape(NW, 2, h), pos.reshape(NW, 2, h))


def _swiglu_body(ble_ref, nv_ref, x_ref, w1_ref, w3_ref, w2_ref, out_ref):
    b = pl.program_id(0)

    @pl.when(b < nv_ref[0])
    def _():
        x = x_ref[...].astype(jnp.bfloat16)
        h1 = jnp.dot(x, w1_ref[0], preferred_element_type=jnp.float32)
        h3 = jnp.dot(x, w3_ref[0], preferred_element_type=jnp.float32)
        h = (h1 * (1.0 / (1.0 + jnp.exp(-h1))) * h3).astype(jnp.bfloat16)
        out_ref[...] = jnp.dot(h, w2_ref[0], preferred_element_type=jnp.float32)


def _swiglu(disp, w1, w3, w2, ble, nvalid):
    grid_spec = pltpu.PrefetchScalarGridSpec(
        num_scalar_prefetch=2,
        grid=(NB,),
        in_specs=[
            pl.BlockSpec((BLK, DIM), lambda b, ble, nv: (b, 0)),
            pl.BlockSpec((1, DIM, FFN), lambda b, ble, nv: (ble[b], 0, 0)),
            pl.BlockSpec((1, DIM, FFN), lambda b, ble, nv: (ble[b], 0, 0)),
            pl.BlockSpec((1, FFN, DIM), lambda b, ble, nv: (ble[b], 0, 0)),
        ],
        out_specs=pl.BlockSpec((BLK, DIM), lambda b, ble, nv: (b, 0)),
    )
    return pl.pallas_call(
        _swiglu_body,
        grid_spec=grid_spec,
        out_shape=jax.ShapeDtypeStruct((PADT, DIM), jnp.float32),
    )(ble, nvalid, disp, w1.astype(jnp.bfloat16), w3.astype(jnp.bfloat16),
      w2.astype(jnp.bfloat16))


def _combine_body(eo_hbm, pos_hbm, gath_hbm, posv, buf, sem):
    w = lax.axis_index("s") * 2 + lax.axis_index("c")
    pltpu.sync_copy(pos_hbm.at[w], posv)
    for k in range(TOP_K):
        pltpu.async_copy(eo_hbm.at[posv.at[k]], buf, sem).wait()
        pltpu.sync_copy(buf, gath_hbm.at[k, pl.ds(w * CH_C, CH_C)])


def _combine_sc(eo, pos_cmb):
    k = functools.partial(
        pl.kernel,
        mesh=plsc.VectorSubcoreMesh(core_axis_name="c", subcore_axis_name="s"),
        out_type=jax.ShapeDtypeStruct((TOP_K, T, DIM), jnp.float32),
        scratch_types=[
            pltpu.VMEM((TOP_K, CH_C), jnp.int32),
            pltpu.VMEM((CH_C, DIM), jnp.float32),
            pltpu.SemaphoreType.DMA,
        ],
    )(_combine_body)
    return k(eo, pos_cmb)


def _wsum_body(g_ref, s_ref, out_ref):
    g = g_ref[...]
    s = s_ref[...]
    out_ref[...] = g[0] * s[:, 0:1] + g[1] * s[:, 1:2]


def _wsum(gath, scores):
    blk = 256
    return pl.pallas_call(
        _wsum_body,
        grid=(T // blk,),
        in_specs=[
            pl.BlockSpec((TOP_K, blk, DIM), lambda b: (0, b, 0)),
            pl.BlockSpec((blk, TOP_K), lambda b: (b, 0)),
        ],
        out_specs=pl.BlockSpec((blk, DIM), lambda b: (b, 0)),
        out_shape=jax.ShapeDtypeStruct((T, DIM), jnp.float32),
    )(gath, scores)


def _bookkeeping(experts):
    flat_e = experts.T.reshape(-1)
    oh = (flat_e[:, None] == jnp.arange(NUM_EXPERTS)[None, :]).astype(jnp.int32)
    csum = jnp.cumsum(oh, axis=0)
    counts = csum[-1]
    rank = jnp.take_along_axis(csum - oh, flat_e[:, None], axis=1)[:, 0]
    pc = ((counts + BLK - 1) // BLK) * BLK
    bounds = jnp.cumsum(pc)
    off = bounds - pc
    pos = (off[flat_e] + rank).astype(jnp.int32)
    tok = (jnp.arange(TT, dtype=jnp.int32) % T)
    nvalid = (bounds[-1] // BLK).astype(jnp.int32).reshape(1)
    ble = jnp.minimum(
        jnp.searchsorted(bounds, jnp.arange(NB) * BLK, side="right"),
        NUM_EXPERTS - 1,
    ).astype(jnp.int32)
    return pos, tok, ble, nvalid


def kernel(hidden_states, gate_w, w1, w2, w3):
    orig_shape = hidden_states.shape
    x = hidden_states.reshape(-1, DIM)

    scores, experts = _router(x, gate_w)
    pos, tok, ble, nvalid = _bookkeeping(experts)

    disp = _dispatch_sc(x, tok, pos)
    eo = _swiglu(disp, w1, w3, w2, ble, nvalid)

    pos_cmb = pos.reshape(TOP_K, NW, CH_C).transpose(1, 0, 2)
    gath = _combine_sc(eo, pos_cmb)
    out = _wsum(gath, scores)
    return out.reshape(orig_shape)


# speedup vs baseline: 1.4849x; 1.4849x over previous
import functools

import jax
import jax.numpy as jnp
from jax import lax
from jax.experimental import pallas as pl
from jax.experimental.pallas import tpu as pltpu
from jax.experimental.pallas import tpu_sc as plsc

NUM_EXPERTS = 8
TOP_K = 2
DIM = 1024
FFN = 2048
T = 2048
TT = TOP_K * T

BLK = 128
PADT = TT + NUM_EXPERTS * BLK
NB = PADT // BLK

NW = 32
CH_D = TT // NW
CH_C = T // NW


def _router_body(x_ref, gw_ref, sc_ref, ex_ref):
    x = x_ref[...]
    gw = gw_ref[...]
    l = lax.dot_general(x, gw, (((1,), (1,)), ((), ())),
                        preferred_element_type=jnp.float32)
    idx = lax.broadcasted_iota(jnp.int32, l.shape, 1)
    m1 = jnp.max(l, axis=1, keepdims=True)
    a1 = jnp.min(jnp.where(l >= m1, idx, NUM_EXPERTS), axis=1)
    neg = jnp.where(idx == a1[:, None], -jnp.inf, l)
    m2 = jnp.max(neg, axis=1, keepdims=True)
    a2 = jnp.min(jnp.where(neg >= m2, idx, NUM_EXPERTS), axis=1)
    den = jnp.sum(jnp.exp(l - m1), axis=1, keepdims=True)
    s1 = 1.0 / den
    s2 = jnp.exp(m2 - m1) / den
    sc_ref[...] = jnp.concatenate([s1, s2], axis=1)
    ex_ref[...] = jnp.concatenate([a1[:, None], a2[:, None]], axis=1)


def _router(x, gate_w):
    blk = 256
    return pl.pallas_call(
        _router_body,
        grid=(T // blk,),
        in_specs=[
            pl.BlockSpec((blk, DIM), lambda b: (b, 0)),
            pl.BlockSpec((NUM_EXPERTS, DIM), lambda b: (0, 0)),
        ],
        out_specs=[
            pl.BlockSpec((blk, TOP_K), lambda b: (b, 0)),
            pl.BlockSpec((blk, TOP_K), lambda b: (b, 0)),
        ],
        out_shape=[
            jax.ShapeDtypeStruct((T, TOP_K), jnp.float32),
            jax.ShapeDtypeStruct((T, TOP_K), jnp.int32),
        ],
    )(x, gate_w)


_DCH = 4


def _dispatch_body(x_hbm, tok_hbm, pos_hbm, disp_hbm, tokv, posv, rows0, rows1,
                   gsem, ssem0, ssem1):
    w = lax.axis_index("s") * 2 + lax.axis_index("c")
    pltpu.sync_copy(tok_hbm.at[w], tokv)
    pltpu.sync_copy(pos_hbm.at[w], posv)
    rows = (rows0, rows1)
    ssems = (ssem0, ssem1)
    scatters = [None, None]
    for j in range(_DCH):
        r = rows[j % 2]
        pltpu.async_copy(x_hbm.at[tokv.at[j]], r, gsem).wait()
        if scatters[j % 2] is not None:
            scatters[j % 2].wait()
        scatters[j % 2] = pltpu.async_copy(r, disp_hbm.at[posv.at[j]],
                                           ssems[j % 2])
    scatters[0].wait()
    scatters[1].wait()


def _dispatch_sc(x, tok, pos):
    h = CH_D // _DCH
    k = functools.partial(
        pl.kernel,
        mesh=plsc.VectorSubcoreMesh(core_axis_name="c", subcore_axis_name="s"),
        out_type=jax.ShapeDtypeStruct((PADT, DIM), jnp.float32),
        scratch_types=[
            pltpu.VMEM((_DCH, h), jnp.int32),
            pltpu.VMEM((_DCH, h), jnp.int32),
            pltpu.VMEM((h, DIM), jnp.float32),
            pltpu.VMEM((h, DIM), jnp.float32),
            pltpu.SemaphoreType.DMA,
            pltpu.SemaphoreType.DMA,
            pltpu.SemaphoreType.DMA,
        ],
    )(_dispatch_body)
    return k(x, tok.reshape(NW, _DCH, h), pos.reshape(NW, _DCH, h))


def _swiglu_body(ble_ref, nv_ref, x_ref, w1_ref, w3_ref, w2_ref, out_ref):
    b = pl.program_id(0)

    @pl.when(b < nv_ref[0])
    def _():
        x = x_ref[...]
        h1 = jnp.dot(x, w1_ref[0], preferred_element_type=jnp.float32)
        h3 = jnp.dot(x, w3_ref[0], preferred_element_type=jnp.float32)
        h = h1 * (1.0 / (1.0 + jnp.exp(-h1))) * h3
        out_ref[...] = jnp.dot(h, w2_ref[0], preferred_element_type=jnp.float32)


def _swiglu(disp, w1, w3, w2, ble, nvalid):
    grid_spec = pltpu.PrefetchScalarGridSpec(
        num_scalar_prefetch=2,
        grid=(NB,),
        in_specs=[
            pl.BlockSpec((BLK, DIM), lambda b, ble, nv: (b, 0)),
            pl.BlockSpec((1, DIM, FFN), lambda b, ble, nv: (ble[b], 0, 0)),
            pl.BlockSpec((1, DIM, FFN), lambda b, ble, nv: (ble[b], 0, 0)),
            pl.BlockSpec((1, FFN, DIM), lambda b, ble, nv: (ble[b], 0, 0)),
        ],
        out_specs=pl.BlockSpec((BLK, DIM), lambda b, ble, nv: (b, 0)),
    )
    return pl.pallas_call(
        _swiglu_body,
        grid_spec=grid_spec,
        out_shape=jax.ShapeDtypeStruct((PADT, DIM), jnp.float32),
    )(ble, nvalid, disp, w1, w3, w2)


def _combine_body(eo_hbm, pos_hbm, gath_hbm, posv, buf, sem):
    w = lax.axis_index("s") * 2 + lax.axis_index("c")
    pltpu.sync_copy(pos_hbm.at[w], posv)
    for k in range(TOP_K):
        pltpu.async_copy(eo_hbm.at[posv.at[k]], buf, sem).wait()
        pltpu.sync_copy(buf, gath_hbm.at[k, pl.ds(w * CH_C, CH_C)])


def _combine_sc(eo, pos_cmb):
    k = functools.partial(
        pl.kernel,
        mesh=plsc.VectorSubcoreMesh(core_axis_name="c", subcore_axis_name="s"),
        out_type=jax.ShapeDtypeStruct((TOP_K, T, DIM), jnp.float32),
        scratch_types=[
            pltpu.VMEM((TOP_K, CH_C), jnp.int32),
            pltpu.VMEM((CH_C, DIM), jnp.float32),
            pltpu.SemaphoreType.DMA,
        ],
    )(_combine_body)
    return k(eo, pos_cmb)


def _wsum_body(g_ref, s_ref, out_ref):
    g = g_ref[...]
    s = s_ref[...]
    out_ref[...] = g[0] * s[:, 0:1] + g[1] * s[:, 1:2]


def _wsum(gath, scores):
    blk = 256
    return pl.pallas_call(
        _wsum_body,
        grid=(T // blk,),
        in_specs=[
            pl.BlockSpec((TOP_K, blk, DIM), lambda b: (0, b, 0)),
            pl.BlockSpec((blk, TOP_K), lambda b: (b, 0)),
        ],
        out_specs=pl.BlockSpec((blk, DIM), lambda b: (b, 0)),
        out_shape=jax.ShapeDtypeStruct((T, DIM), jnp.float32),
    )(gath, scores)


def _bookkeeping(experts):
    nch, ch = 32, TT // 32
    flat_e = experts.T.reshape(-1)
    oh = (flat_e[:, None] == jnp.arange(NUM_EXPERTS)[None, :]).astype(
        jnp.float32)
    oh3 = oh.reshape(nch, ch, NUM_EXPERTS)
    tril = jnp.tril(jnp.ones((ch, ch), jnp.float32))
    local = jnp.einsum("ij,bjk->bik", tril, oh3,
                       precision=jax.lax.Precision.HIGHEST)
    sums = oh3.sum(axis=1)
    trilc = jnp.tril(jnp.ones((nch, nch), jnp.float32), k=-1)
    pref = trilc @ sums
    csum = (local + pref[:, None, :]).reshape(TT, NUM_EXPERTS)
    counts = sums.sum(axis=0)
    rank = (csum * oh).sum(axis=1) - 1.0
    pc = jnp.ceil(counts / BLK) * BLK
    bounds = jnp.cumsum(pc)
    off = bounds - pc
    pos = ((off[None, :] * oh).sum(axis=1) + rank).astype(jnp.int32)
    tok = (jnp.arange(TT, dtype=jnp.int32) % T)
    nvalid = (bounds[-1] / BLK).astype(jnp.int32).reshape(1)
    starts = (jnp.arange(NB) * BLK).astype(jnp.float32)
    ble = jnp.minimum(
        (starts[:, None] >= bounds[None, :]).sum(axis=1), NUM_EXPERTS - 1
    ).astype(jnp.int32)
    return pos, tok, ble, nvalid


def kernel(hidden_states, gate_w, w1, w2, w3):
    orig_shape = hidden_states.shape
    x = hidden_states.reshape(-1, DIM)

    scores, experts = _router(x, gate_w)
    pos, tok, ble, nvalid = _bookkeeping(experts)

    disp = _dispatch_sc(x, tok, pos)
    eo = _swiglu(disp, w1, w3, w2, ble, nvalid)

    pos_cmb = pos.reshape(TOP_K, NW, CH_C).transpose(1, 0, 2)
    gath = _combine_sc(eo, pos_cmb)
    out = _wsum(gath, scores)
    return out.reshape(orig_shape)


# speedup vs baseline: 1.5199x; 1.0236x over previous
import functools

import jax
import jax.numpy as jnp
from jax import lax
from jax.experimental import pallas as pl
from jax.experimental.pallas import tpu as pltpu
from jax.experimental.pallas import tpu_sc as plsc

NUM_EXPERTS = 8
TOP_K = 2
DIM = 1024
FFN = 2048
T = 2048
TT = TOP_K * T

BLK = 128
PADT = TT + NUM_EXPERTS * BLK
NB = PADT // BLK

NW = 32
CH_D = TT // NW
CH_C = T // NW

_NCH = T // BLK


def _router_body(x_ref, gw_ref, sc_ref, pd_ref, meta_ref):
    x = x_ref[...]
    gw = gw_ref[...]
    l = lax.dot_general(x, gw, (((1,), (1,)), ((), ())),
                        preferred_element_type=jnp.float32)
    idx = lax.broadcasted_iota(jnp.int32, l.shape, 1)
    m1 = jnp.max(l, axis=1, keepdims=True)
    a1 = jnp.min(jnp.where(l >= m1, idx, NUM_EXPERTS), axis=1)
    neg = jnp.where(idx == a1[:, None], -jnp.inf, l)
    m2 = jnp.max(neg, axis=1, keepdims=True)
    a2 = jnp.min(jnp.where(neg >= m2, idx, NUM_EXPERTS), axis=1)
    den = jnp.sum(jnp.exp(l - m1), axis=1, keepdims=True)
    s1 = 1.0 / den
    s2 = jnp.exp(m2 - m1) / den
    sc_ref[...] = jnp.concatenate([s1, s2], axis=1)

    oh1 = (idx == a1[:, None]).astype(jnp.float32)
    oh2 = (idx == a2[:, None]).astype(jnp.float32)
    tr = lax.broadcasted_iota(jnp.int32, (BLK, BLK), 0)
    tc = lax.broadcasted_iota(jnp.int32, (BLK, BLK), 1)
    tril = (tr >= tc).astype(jnp.float32)
    run = jnp.zeros((1, NUM_EXPERTS), jnp.float32)
    cs = []
    for oh in (oh1, oh2):
        for i in range(_NCH):
            chunk = lax.slice(oh, (i * BLK, 0), ((i + 1) * BLK, NUM_EXPERTS))
            c = lax.dot_general(tril, chunk, (((1,), (0,)), ((), ())),
                                preferred_element_type=jnp.float32) + run
            cs.append(c)
            run = lax.slice(c, (BLK - 1, 0), (BLK, NUM_EXPERTS))
    csum1 = jnp.concatenate(cs[:_NCH], axis=0)
    csum2 = jnp.concatenate(cs[_NCH:], axis=0)
    counts = run

    pc = jnp.ceil(counts * (1.0 / BLK)) * BLK
    ur = lax.broadcasted_iota(jnp.int32, (NUM_EXPERTS, NUM_EXPERTS), 0)
    uc = lax.broadcasted_iota(jnp.int32, (NUM_EXPERTS, NUM_EXPERTS), 1)
    triu = (ur <= uc).astype(jnp.float32)
    bounds = lax.dot_general(pc, triu, (((1,), (0,)), ((), ())),
                             preferred_element_type=jnp.float32)
    off = bounds - pc

    pos1 = (off * oh1).sum(axis=1) + (csum1 * oh1).sum(axis=1) - 1.0
    pos2 = (off * oh2).sum(axis=1) + (csum2 * oh2).sum(axis=1) - 1.0
    pd = jnp.concatenate(
        [pos1.reshape(_NCH, BLK), pos2.reshape(_NCH, BLK)], axis=0)
    pd_ref[...] = pd.astype(jnp.int32)

    rows = lax.broadcasted_iota(
        jnp.int32, (BLK, NUM_EXPERTS), 0).astype(jnp.float32) * BLK
    blef = jnp.minimum(
        jnp.sum((rows >= bounds).astype(jnp.float32), axis=1),
        NUM_EXPERTS - 1.0)
    nv = bounds[0, NUM_EXPERTS - 1] * (1.0 / BLK)
    lane = lax.broadcasted_iota(jnp.int32, (BLK,), 0)
    meta = jnp.where(lane == BLK - 1, nv, blef)
    meta_ref[...] = meta.reshape(1, BLK).astype(jnp.int32)


def _router(x, gate_w):
    return pl.pallas_call(
        _router_body,
        out_shape=[
            jax.ShapeDtypeStruct((T, TOP_K), jnp.float32),
            jax.ShapeDtypeStruct((NW, CH_D), jnp.int32),
            jax.ShapeDtypeStruct((1, BLK), jnp.int32),
        ],
    )(x, gate_w)


_DCH = 4
_H = CH_D // _DCH


def _dispatch_body(x_hbm, pos_hbm, disp_hbm, posv, rows0, rows1,
                   gsem, ssem0, ssem1):
    w = lax.axis_index("s") * 2 + lax.axis_index("c")
    pltpu.sync_copy(pos_hbm.at[w], posv)
    base = (w % (T // CH_D)) * CH_D
    rows = (rows0, rows1)
    ssems = (ssem0, ssem1)
    scatters = [None, None]
    for j in range(_DCH):
        r = rows[j % 2]
        pltpu.async_copy(x_hbm.at[pl.ds(base + j * _H, _H)], r, gsem).wait()
        if scatters[j % 2] is not None:
            scatters[j % 2].wait()
        scatters[j % 2] = pltpu.async_copy(r, disp_hbm.at[posv.at[j]],
                                           ssems[j % 2])
    scatters[0].wait()
    scatters[1].wait()


def _dispatch_sc(x, pos):
    k = functools.partial(
        pl.kernel,
        mesh=plsc.VectorSubcoreMesh(core_axis_name="c", subcore_axis_name="s"),
        out_type=jax.ShapeDtypeStruct((PADT, DIM), jnp.float32),
        scratch_types=[
            pltpu.VMEM((_DCH, _H), jnp.int32),
            pltpu.VMEM((_H, DIM), jnp.float32),
            pltpu.VMEM((_H, DIM), jnp.float32),
            pltpu.SemaphoreType.DMA,
            pltpu.SemaphoreType.DMA,
            pltpu.SemaphoreType.DMA,
        ],
    )(_dispatch_body)
    return k(x, pos.reshape(NW, _DCH, _H))


def _swiglu_body(ble_ref, nv_ref, x_ref, w1_ref, w3_ref, w2_ref, out_ref):
    b = pl.program_id(0)

    @pl.when(b < nv_ref[0])
    def _():
        x = x_ref[...]
        h1 = jnp.dot(x, w1_ref[0], preferred_element_type=jnp.float32)
        h3 = jnp.dot(x, w3_ref[0], preferred_element_type=jnp.float32)
        h = h1 * (1.0 / (1.0 + jnp.exp(-h1))) * h3
        out_ref[...] = jnp.dot(h, w2_ref[0], preferred_element_type=jnp.float32)


def _swiglu(disp, w1, w3, w2, ble, nvalid):
    grid_spec = pltpu.PrefetchScalarGridSpec(
        num_scalar_prefetch=2,
        grid=(NB,),
        in_specs=[
            pl.BlockSpec((BLK, DIM), lambda b, ble, nv: (b, 0)),
            pl.BlockSpec((1, DIM, FFN), lambda b, ble, nv: (ble[b], 0, 0)),
            pl.BlockSpec((1, DIM, FFN), lambda b, ble, nv: (ble[b], 0, 0)),
            pl.BlockSpec((1, FFN, DIM), lambda b, ble, nv: (ble[b], 0, 0)),
        ],
        out_specs=pl.BlockSpec((BLK, DIM), lambda b, ble, nv: (b, 0)),
    )
    return pl.pallas_call(
        _swiglu_body,
        grid_spec=grid_spec,
        out_shape=jax.ShapeDtypeStruct((PADT, DIM), jnp.float32),
    )(ble, nvalid, disp, w1, w3, w2)


def _combine_body(eo_hbm, pos_hbm, gath_hbm, posv, buf, sem):
    w = lax.axis_index("s") * 2 + lax.axis_index("c")
    for k in range(TOP_K):
        pltpu.sync_copy(pos_hbm.at[k, w], posv.at[k])
    for k in range(TOP_K):
        pltpu.async_copy(eo_hbm.at[posv.at[k]], buf, sem).wait()
        pltpu.sync_copy(buf, gath_hbm.at[k, pl.ds(w * CH_C, CH_C)])


def _combine_sc(eo, pos_cmb):
    k = functools.partial(
        pl.kernel,
        mesh=plsc.VectorSubcoreMesh(core_axis_name="c", subcore_axis_name="s"),
        out_type=jax.ShapeDtypeStruct((TOP_K, T, DIM), jnp.float32),
        scratch_types=[
            pltpu.VMEM((TOP_K, CH_C), jnp.int32),
            pltpu.VMEM((CH_C, DIM), jnp.float32),
            pltpu.SemaphoreType.DMA,
        ],
    )(_combine_body)
    return k(eo, pos_cmb)


def _wsum_body(g_ref, s_ref, out_ref):
    g = g_ref[...]
    s = s_ref[...]
    out_ref[...] = g[0] * s[:, 0:1] + g[1] * s[:, 1:2]


def _wsum(gath, scores):
    blk = 256
    return pl.pallas_call(
        _wsum_body,
        grid=(T // blk,),
        in_specs=[
            pl.BlockSpec((TOP_K, blk, DIM), lambda b: (0, b, 0)),
            pl.BlockSpec((blk, TOP_K), lambda b: (b, 0)),
        ],
        out_specs=pl.BlockSpec((blk, DIM), lambda b: (b, 0)),
        out_shape=jax.ShapeDtypeStruct((T, DIM), jnp.float32),
    )(gath, scores)


def kernel(hidden_states, gate_w, w1, w2, w3):
    orig_shape = hidden_states.shape
    x = hidden_states.reshape(-1, DIM)

    scores, pd, meta = _router(x, gate_w)
    ble = meta[0, :NB]
    nvalid = meta[0, BLK - 1:BLK]

    disp = _dispatch_sc(x, pd)
    eo = _swiglu(disp, w1, w3, w2, ble, nvalid)

    gath = _combine_sc(eo, pd.reshape(TOP_K, NW, CH_C))
    out = _wsum(gath, scores)
    return out.reshape(orig_shape)


# speedup vs baseline: 1.5406x; 1.0136x over previous
import functools

import jax
import jax.numpy as jnp
from jax import lax
from jax.experimental import pallas as pl
from jax.experimental.pallas import tpu as pltpu
from jax.experimental.pallas import tpu_sc as plsc

NUM_EXPERTS = 8
TOP_K = 2
DIM = 1024
FFN = 2048
T = 2048
TT = TOP_K * T

BLK = 128
PADT = TT + NUM_EXPERTS * BLK
NB = PADT // BLK

NW = 32
CH_D = TT // NW
CH_C = T // NW

_NCH = T // BLK


def _router_body(x_ref, gw_ref, sc_ref, pd_ref, meta_ref, ssrc_ref):
    x = x_ref[...]
    gw = gw_ref[...]
    l = lax.dot_general(x, gw, (((1,), (1,)), ((), ())),
                        preferred_element_type=jnp.float32)
    idx = lax.broadcasted_iota(jnp.int32, l.shape, 1)
    m1 = jnp.max(l, axis=1, keepdims=True)
    a1 = jnp.min(jnp.where(l >= m1, idx, NUM_EXPERTS), axis=1)
    neg = jnp.where(idx == a1[:, None], -jnp.inf, l)
    m2 = jnp.max(neg, axis=1, keepdims=True)
    a2 = jnp.min(jnp.where(neg >= m2, idx, NUM_EXPERTS), axis=1)
    den = jnp.sum(jnp.exp(l - m1), axis=1, keepdims=True)
    s1 = 1.0 / den
    s2 = jnp.exp(m2 - m1) / den
    sc_ref[...] = jnp.concatenate([s1, s2], axis=1)
    ssrc_ref[...] = jnp.broadcast_to(
        jnp.concatenate([s1, s2], axis=0), (TT, 128))

    oh1 = (idx == a1[:, None]).astype(jnp.float32)
    oh2 = (idx == a2[:, None]).astype(jnp.float32)
    tr = lax.broadcasted_iota(jnp.int32, (BLK, BLK), 0)
    tc = lax.broadcasted_iota(jnp.int32, (BLK, BLK), 1)
    tril = (tr >= tc).astype(jnp.float32)
    run = jnp.zeros((1, NUM_EXPERTS), jnp.float32)
    cs = []
    for oh in (oh1, oh2):
        for i in range(_NCH):
            chunk = lax.slice(oh, (i * BLK, 0), ((i + 1) * BLK, NUM_EXPERTS))
            c = lax.dot_general(tril, chunk, (((1,), (0,)), ((), ())),
                                preferred_element_type=jnp.float32) + run
            cs.append(c)
            run = lax.slice(c, (BLK - 1, 0), (BLK, NUM_EXPERTS))
    csum1 = jnp.concatenate(cs[:_NCH], axis=0)
    csum2 = jnp.concatenate(cs[_NCH:], axis=0)
    counts = run

    pc = jnp.ceil(counts * (1.0 / BLK)) * BLK
    ur = lax.broadcasted_iota(jnp.int32, (NUM_EXPERTS, NUM_EXPERTS), 0)
    uc = lax.broadcasted_iota(jnp.int32, (NUM_EXPERTS, NUM_EXPERTS), 1)
    triu = (ur <= uc).astype(jnp.float32)
    bounds = lax.dot_general(pc, triu, (((1,), (0,)), ((), ())),
                             preferred_element_type=jnp.float32)
    off = bounds - pc

    pos1 = (off * oh1).sum(axis=1) + (csum1 * oh1).sum(axis=1) - 1.0
    pos2 = (off * oh2).sum(axis=1) + (csum2 * oh2).sum(axis=1) - 1.0
    pd = jnp.concatenate(
        [pos1.reshape(_NCH, BLK), pos2.reshape(_NCH, BLK)], axis=0)
    pd_ref[...] = pd.astype(jnp.int32)

    rows = lax.broadcasted_iota(
        jnp.int32, (BLK, NUM_EXPERTS), 0).astype(jnp.float32) * BLK
    blef = jnp.minimum(
        jnp.sum((rows >= bounds).astype(jnp.float32), axis=1),
        NUM_EXPERTS - 1.0)
    nv = bounds[0, NUM_EXPERTS - 1] * (1.0 / BLK)
    lane = lax.broadcasted_iota(jnp.int32, (BLK,), 0)
    meta = jnp.where(lane == BLK - 1, nv, blef)
    meta_ref[...] = meta.reshape(1, BLK).astype(jnp.int32)


def _router(x, gate_w):
    return pl.pallas_call(
        _router_body,
        out_shape=[
            jax.ShapeDtypeStruct((T, TOP_K), jnp.float32),
            jax.ShapeDtypeStruct((NW, CH_D), jnp.int32),
            jax.ShapeDtypeStruct((1, BLK), jnp.int32),
            jax.ShapeDtypeStruct((TT, 128), jnp.float32),
        ],
    )(x, gate_w)


_DCH = 4
_H = CH_D // _DCH


def _dispatch_body(x_hbm, pos_hbm, ssrc_hbm, disp_hbm, dscore_hbm,
                   posv, svmem, rows0, rows1, gsem, ssem0, ssem1):
    w = lax.axis_index("s") * 2 + lax.axis_index("c")
    pltpu.sync_copy(pos_hbm.at[w], posv)
    pltpu.sync_copy(ssrc_hbm.at[w], svmem)
    base = (w % (T // CH_D)) * CH_D
    rows = (rows0, rows1)
    ssems = (ssem0, ssem1)
    scatters = [None, None]
    for j in range(_DCH):
        r = rows[j % 2]
        pltpu.async_copy(x_hbm.at[pl.ds(base + j * _H, _H)], r, gsem).wait()
        if scatters[j % 2] is not None:
            scatters[j % 2].wait()
        scatters[j % 2] = pltpu.async_copy(r, disp_hbm.at[posv.at[j]],
                                           ssems[j % 2])
        pltpu.sync_copy(svmem.at[j], dscore_hbm.at[posv.at[j]])
    scatters[0].wait()
    scatters[1].wait()


def _dispatch_sc(x, pos, ssrc):
    k = functools.partial(
        pl.kernel,
        mesh=plsc.VectorSubcoreMesh(core_axis_name="c", subcore_axis_name="s"),
        out_type=[
            jax.ShapeDtypeStruct((PADT, DIM), jnp.float32),
            jax.ShapeDtypeStruct((PADT, 128), jnp.float32),
        ],
        scratch_types=[
            pltpu.VMEM((_DCH, _H), jnp.int32),
            pltpu.VMEM((_DCH, _H, 128), jnp.float32),
            pltpu.VMEM((_H, DIM), jnp.float32),
            pltpu.VMEM((_H, DIM), jnp.float32),
            pltpu.SemaphoreType.DMA,
            pltpu.SemaphoreType.DMA,
            pltpu.SemaphoreType.DMA,
        ],
    )(_dispatch_body)
    return k(x, pos.reshape(NW, _DCH, _H), ssrc.reshape(NW, _DCH, _H, 128))


def _swiglu_body(meta_ref, x_ref, ds_ref, w1_ref, w3_ref, w2_ref, out_ref):
    b = pl.program_id(0)

    @pl.when(b < meta_ref[0, BLK - 1])
    def _():
        x = x_ref[...]
        h1 = jnp.dot(x, w1_ref[0], preferred_element_type=jnp.float32)
        h3 = jnp.dot(x, w3_ref[0], preferred_element_type=jnp.float32)
        h = h1 * (1.0 / (1.0 + jnp.exp(-h1))) * h3
        o = jnp.dot(h, w2_ref[0], preferred_element_type=jnp.float32)
        out_ref[...] = o * ds_ref[:, 0:1]


def _swiglu(disp, dscore, w1, w3, w2, meta):
    grid_spec = pltpu.PrefetchScalarGridSpec(
        num_scalar_prefetch=1,
        grid=(NB,),
        in_specs=[
            pl.BlockSpec((BLK, DIM), lambda b, m: (b, 0)),
            pl.BlockSpec((BLK, 128), lambda b, m: (b, 0)),
            pl.BlockSpec((1, DIM, FFN), lambda b, m: (m[0, b], 0, 0)),
            pl.BlockSpec((1, DIM, FFN), lambda b, m: (m[0, b], 0, 0)),
            pl.BlockSpec((1, FFN, DIM), lambda b, m: (m[0, b], 0, 0)),
        ],
        out_specs=pl.BlockSpec((BLK, DIM), lambda b, m: (b, 0)),
    )
    return pl.pallas_call(
        _swiglu_body,
        grid_spec=grid_spec,
        out_shape=jax.ShapeDtypeStruct((PADT, DIM), jnp.float32),
    )(meta, disp, dscore, w1, w3, w2)


_NS = 16


_HC = CH_C // 2


def _combine_body(eo_hbm, pos_hbm, out_hbm, posv, buf0, buf1, sem0, sem1):
    c = lax.axis_index("c")
    s = lax.axis_index("s")
    v = s * 2 + c
    for k in range(TOP_K):
        for h in range(2):
            pltpu.sync_copy(pos_hbm.at[k, v, h], posv.at[k, h])
    for h in range(2):
        cp0 = pltpu.async_copy(eo_hbm.at[posv.at[0, h]], buf0, sem0)
        cp1 = pltpu.async_copy(eo_hbm.at[posv.at[1, h]], buf1, sem1)
        cp0.wait()
        cp1.wait()

        def _row(r, carry):
            for q in range(DIM // 16):
                plsc.addupdate(buf0.at[r, pl.ds(q * 16, 16)],
                               buf1[r, pl.ds(q * 16, 16)])
            return carry

        lax.fori_loop(0, _HC, _row, 0)
        pltpu.sync_copy(buf0, out_hbm.at[pl.ds(v * CH_C + h * _HC, _HC)])


def _combine_sc(eo, pos_cmb):
    k = functools.partial(
        pl.kernel,
        mesh=plsc.VectorSubcoreMesh(core_axis_name="c", subcore_axis_name="s"),
        out_type=jax.ShapeDtypeStruct((T, DIM), jnp.float32),
        scratch_types=[
            pltpu.VMEM((TOP_K, 2, _HC), jnp.int32),
            pltpu.VMEM((_HC, DIM), jnp.float32),
            pltpu.VMEM((_HC, DIM), jnp.float32),
            pltpu.SemaphoreType.DMA,
            pltpu.SemaphoreType.DMA,
        ],
    )(_combine_body)
    return k(eo, pos_cmb)


def kernel(hidden_states, gate_w, w1, w2, w3):
    orig_shape = hidden_states.shape
    x = hidden_states.reshape(-1, DIM)

    scores, pd, meta, ssrc = _router(x, gate_w)
    del scores

    disp, dscore = _dispatch_sc(x, pd, ssrc)
    eo = _swiglu(disp, dscore, w1, w3, w2, meta)

    out = _combine_sc(eo, pd.reshape(TOP_K, T // CH_C, 2, _HC))
    return out.reshape(orig_shape)
